# Initial kernel scaffold; baseline (speedup 1.0000x reference)
import jax
import jax.numpy as jnp
from jax.experimental import pallas as pl


def kernel(x, hashtable):
    raise NotImplementedError("write your pallas kernel here")


# speedup vs baseline: 1.9958x; 1.9958x over previous
import functools

import jax
import jax.numpy as jnp
from jax import lax
from jax.experimental import pallas as pl
from jax.experimental.pallas import tpu as pltpu
from jax.experimental.pallas import tpu_sc as plsc

_L = 16
_T = 2 ** 19
_F = 2
_N_MIN = 16
_N_MAX = 4096
_MASK = _T - 1
_PI1 = -1640531535
_PI2 = 805459861

_LANES = 16
_NC = 2
_NS = 16
_NW = _NC * _NS
_RW = 16


def _build(N):
    P = N // _NW
    SB = 4
    Q = P // SB
    GQ = Q // _LANES
    NIQ = 8 * Q
    CH = 128
    NCH = NIQ // CH
    KOUT = 8

    mesh = plsc.VectorSubcoreMesh(core_axis_name="c", subcore_axis_name="s")

    @functools.partial(
        pl.kernel,
        out_type=jax.ShapeDtypeStruct((2 * _L * N,), jnp.float32),
        mesh=mesh,
        compiler_params=pltpu.CompilerParams(needs_layout_passes=False,
                                             use_tc_tiling_on_sc=False),
        scratch_types=[
            pltpu.VMEM((3 * P,), jnp.float32),
            pltpu.VMEM((3 * P,), jnp.float32),
            pltpu.VMEM((_L * _LANES,), jnp.float32),
            pltpu.VMEM((NIQ,), jnp.int32),
            pltpu.VMEM((NIQ,), jnp.int32),
            pltpu.VMEM((NIQ, _RW), jnp.float32),
            pltpu.VMEM((2 * P,), jnp.float32),
            pltpu.SemaphoreType.DMA,
        ],
    )
    def enc(x_hbm, nl_hbm, table_hbm, out_hbm,
            xs_v, fr_v, nl_v, idx_v, col_v, rows_v, o_v, sem):
        wid = lax.axis_index("s") * _NC + lax.axis_index("c")
        base = wid * P
        for d in range(3):
            pltpu.sync_copy(x_hbm.at[pl.ds(d * N + base, P)],
                            xs_v.at[pl.ds(d * P, P)])
        pltpu.sync_copy(nl_hbm, nl_v)

        iota = lax.iota(jnp.int32, _LANES)

        def level_body(l, _):
            nl = nl_v[pl.ds(l * _LANES, _LANES)]
            lR = l * (_T // 8)

            def sb_body(qb, _):
                qo = qb * Q

                def idx_body(g, _):
                    o = qo + g * _LANES
                    xn0 = xs_v[pl.ds(o, _LANES)] * nl
                    xn1 = xs_v[pl.ds(P + o, _LANES)] * nl
                    xn2 = xs_v[pl.ds(2 * P + o, _LANES)] * nl
                    lb0 = xn0.astype(jnp.int32)
                    lb1 = xn1.astype(jnp.int32)
                    lb2 = xn2.astype(jnp.int32)
                    fr_v[pl.ds(o, _LANES)] = xn0 - lb0.astype(jnp.float32)
                    fr_v[pl.ds(P + o, _LANES)] = xn1 - lb1.astype(jnp.float32)
                    fr_v[pl.ds(2 * P + o, _LANES)] = xn2 - lb2.astype(jnp.float32)
                    a1 = lb1 * _PI1
                    a2 = lb2 * _PI2
                    b0 = lb0 + 1
                    b1 = a1 + _PI1
                    b2 = a2 + _PI2
                    for c in range(8):
                        h0 = b0 if (c >> 2) & 1 else lb0
                        h1 = b1 if (c >> 1) & 1 else a1
                        h2 = b2 if c & 1 else a2
                        h = (h0 ^ h1 ^ h2) & _MASK
                        co = pl.ds(c * Q + g * _LANES, _LANES)
                        idx_v[co] = (h >> 3) + lR
                        col_v[co] = (h & 7) << 1

                lax.fori_loop(0, GQ, idx_body, None)

                def dma_body(jj, _):
                    hs = []
                    for j2 in range(KOUT):
                        ch = jj * KOUT + j2
                        hs.append(pltpu.async_copy(
                            table_hbm.at[idx_v.at[pl.ds(ch * CH, CH)]],
                            rows_v.at[pl.ds(ch * CH, CH)], sem))
                    for h in hs:
                        h.wait()

                lax.fori_loop(0, NCH // KOUT, dma_body, None)

                def interp_body(g, _):
                    o = qo + g * _LANES
                    f0 = fr_v[pl.ds(o, _LANES)]
                    f1 = fr_v[pl.ds(P + o, _LANES)]
                    f2 = fr_v[pl.ds(2 * P + o, _LANES)]
                    g0 = 1.0 - f0
                    g1 = 1.0 - f1
                    g2 = 1.0 - f2
                    pair = (g0 * g1, g0 * f1, f0 * g1, f0 * f1)
                    rid0 = g * _LANES + iota
                    acc0 = jnp.zeros((_LANES,), jnp.float32)
                    acc1 = jnp.zeros((_LANES,), jnp.float32)
                    for c in range(8):
                        w = pair[c >> 1] * (f2 if c & 1 else g2)
                        rid = rid0 + c * Q
                        col = col_v[pl.ds(c * Q + g * _LANES, _LANES)]
                        e0 = plsc.load_gather(rows_v, [rid, col])
                        e1 = plsc.load_gather(rows_v, [rid, col + 1])
                        acc0 = acc0 + w * e0
                        acc1 = acc1 + w * e1
                    o_v[pl.ds(o, _LANES)] = acc0
                    o_v[pl.ds(P + o, _LANES)] = acc1

                lax.fori_loop(0, GQ, interp_body, None)

            lax.fori_loop(0, SB, sb_body, None)
            pltpu.sync_copy(o_v.at[pl.ds(0, P)],
                            out_hbm.at[pl.ds(2 * l * N + base, P)])
            pltpu.sync_copy(o_v.at[pl.ds(P, P)],
                            out_hbm.at[pl.ds((2 * l + 1) * N + base, P)])

        lax.fori_loop(0, _L, level_body, None)

    return enc


def kernel(x, hashtable):
    N = x.shape[0]
    b = jnp.exp(jnp.log(_N_MAX / _N_MIN) / (_L - 1))
    n_levels = jnp.floor(_N_MIN * b ** jnp.arange(_L))
    nl_b = jnp.broadcast_to(n_levels[:, None].astype(jnp.float32),
                            (_L, _LANES)).reshape(-1)
    xt = x.T.reshape(-1)
    table = hashtable.reshape(_L * _T * _F // _RW, _RW)
    y = _build(N)(xt, nl_b, table)
    return y.reshape(2 * _L, N).T
